# Initial kernel scaffold; baseline (speedup 1.0000x reference)
import jax
import jax.numpy as jnp
from jax.experimental import pallas as pl


def kernel(x, edge_index, batch, params):
    raise NotImplementedError("write your pallas kernel here")


# speedup vs baseline: 1.5457x; 1.5457x over previous
import functools

import jax
import jax.numpy as jnp
from jax import lax
from jax.experimental import pallas as pl
from jax.experimental.pallas import tpu as pltpu
from jax.experimental.pallas import tpu_sc as plsc

NUM_RUNS = 4
P_DROP = 0.1
EB = 128
NSC = 2
NTEC = 16
RB = 400

_f32 = jnp.float32


def _make_segment_sum(RN, W, RE_pad, dump_rows):
    HALF = RN // NSC
    ZROWS = (HALF + dump_rows + NTEC * 8 - 1) // (NTEC * 8) * 8
    ACC = ZROWS * NTEC
    OUTR = HALF // NTEC // 8 * 8
    REM = HALF - OUTR * NTEC
    stripe = RE_pad // NTEC
    NB = stripe // EB

    mesh = plsc.VectorSubcoreMesh(
        core_axis_name="c", subcore_axis_name="s",
        num_cores=NSC, num_subcores=NTEC)

    def body(rsrc, rdst, zrows, h0, h1, h2, h3,
             o0, o1, o2, o3, sidx, didx, rows, acc, gsem):
        c = lax.axis_index("c")
        s = lax.axis_index("s")
        base_row = c * HALF
        hs = [h0, h1, h2, h3]
        os_ = [o0, o1, o2, o3]
        for f in range(4):
            pltpu.sync_copy(zrows, acc.at[pl.ds(s * ZROWS, ZROWS)])
            plsc.subcore_barrier()

            def step(b, carry):
                base = s * stripe + b * EB
                pltpu.sync_copy(rsrc.at[pl.ds(base, EB)], sidx)
                pltpu.sync_copy(rdst.at[pl.ds(base, EB)], didx)
                for j in range(EB // 16):
                    v = didx[pl.ds(j * 16, 16)]
                    rel = v - base_row
                    ok = (rel >= 0) & (rel < HALF)
                    didx[pl.ds(j * 16, 16)] = jnp.where(ok, rel, HALF)
                pltpu.async_copy(hs[f].at[sidx], rows, gsem).wait()
                pltpu.sync_copy(rows, acc.at[didx], add=True)
                return carry

            lax.fori_loop(0, NB, step, 0, unroll=False)
            plsc.subcore_barrier()
            pltpu.sync_copy(
                acc.at[pl.ds(s * OUTR, OUTR)],
                os_[f].at[pl.ds(base_row + s * OUTR, OUTR)])
            if REM:
                @pl.when(s == 0)
                def _():
                    pltpu.sync_copy(
                        acc.at[pl.ds(NTEC * OUTR, REM)],
                        os_[f].at[pl.ds(base_row + NTEC * OUTR, REM)])
            plsc.subcore_barrier()

    out = [jax.ShapeDtypeStruct((RN, W), _f32)] * 4
    return pl.kernel(
        body, out_type=out, mesh=mesh,
        compiler_params=pltpu.CompilerParams(use_tc_tiling_on_sc=False),
        scratch_types=[
            pltpu.VMEM((EB,), jnp.int32),
            pltpu.VMEM((EB,), jnp.int32),
            pltpu.VMEM((EB, W), _f32),
            pltpu.VMEM_SHARED((ACC, W), _f32),
            pltpu.SemaphoreType.DMA,
        ],
        name=f"gin_segsum_w{W}")


def _drop_expand(x, keep, W):
    n, fin = x.shape
    nb = n // RB

    def body(x_ref, k_ref, *outs):
        xb = x_ref[...] * k_ref[0]
        for j in range(4):
            outs[j][...] = xb[:, j * W:(j + 1) * W]

    grid = (NUM_RUNS, nb)
    return pl.pallas_call(
        body,
        grid=grid,
        in_specs=[
            pl.BlockSpec((RB, fin), lambda r, i: (i, 0)),
            pl.BlockSpec((1, RB, 1), lambda r, i: (r, i, 0)),
        ],
        out_specs=[pl.BlockSpec((RB, W), lambda r, i: (r * (n // RB) + i, 0))
                   for _ in range(4)],
        out_shape=[jax.ShapeDtypeStruct((NUM_RUNS * n, W), _f32)] * 4,
        name="drop_expand",
    )(x, keep)


def _mm1_stats(hs, aggs, w1, b1):
    RN = hs[0].shape[0]
    W = hs[0].shape[1]
    fin = 4 * W
    dim = w1.shape[1]
    nb = RN // RB

    def body(h0, h1, h2, h3, a0, a1, a2, a3, w_ref, b_ref, y_ref, s_ref, q_ref):
        i = pl.program_id(0)
        hh = jnp.concatenate([h0[...], h1[...], h2[...], h3[...]], axis=1)
        aa = jnp.concatenate([a0[...], a1[...], a2[...], a3[...]], axis=1)
        y = lax.dot_general((hh + aa), w_ref[...], (((1,), (0,)), ((), ())),
                            preferred_element_type=_f32) + b_ref[...]
        y_ref[...] = y
        ps = y.reshape(RB // 8, 8, dim).sum(axis=0)
        pq = (y * y).reshape(RB // 8, 8, dim).sum(axis=0)

        @pl.when(i == 0)
        def _():
            s_ref[...] = ps
            q_ref[...] = pq

        @pl.when(i > 0)
        def _():
            s_ref[...] += ps
            q_ref[...] += pq

    cspec = [pl.BlockSpec((RB, W), lambda i: (i, 0)) for _ in range(8)]
    return pl.pallas_call(
        body,
        grid=(nb,),
        in_specs=cspec + [
            pl.BlockSpec((fin, dim), lambda i: (0, 0)),
            pl.BlockSpec((1, dim), lambda i: (0, 0)),
        ],
        out_specs=[
            pl.BlockSpec((RB, dim), lambda i: (i, 0)),
            pl.BlockSpec((8, dim), lambda i: (0, 0)),
            pl.BlockSpec((8, dim), lambda i: (0, 0)),
        ],
        out_shape=[
            jax.ShapeDtypeStruct((RN, dim), _f32),
            jax.ShapeDtypeStruct((8, dim), _f32),
            jax.ShapeDtypeStruct((8, dim), _f32),
        ],
        name="gin_mm1",
    )(*hs, *aggs, w1, b1)


def _bn_relu_mm2(y1, s1, q1, g1, bb1, w2, b2):
    RN, dim = y1.shape
    nb = RN // RB
    inv_n = 1.0 / RN

    def body(y_ref, s_ref, q_ref, g_ref, bb_ref, w_ref, b_ref,
             o_ref, so_ref, qo_ref):
        i = pl.program_id(0)
        mu = s_ref[...].sum(axis=0, keepdims=True) * inv_n
        var = q_ref[...].sum(axis=0, keepdims=True) * inv_n - mu * mu
        scale = g_ref[...] * lax.rsqrt(var + 1e-5)
        a = jnp.maximum((y_ref[...] - mu) * scale + bb_ref[...], 0.0)
        y = lax.dot_general(a, w_ref[...], (((1,), (0,)), ((), ())),
                            preferred_element_type=_f32) + b_ref[...]
        o_ref[...] = y
        ps = y.reshape(RB // 8, 8, dim).sum(axis=0)
        pq = (y * y).reshape(RB // 8, 8, dim).sum(axis=0)

        @pl.when(i == 0)
        def _():
            so_ref[...] = ps
            qo_ref[...] = pq

        @pl.when(i > 0)
        def _():
            so_ref[...] += ps
            qo_ref[...] += pq

    return pl.pallas_call(
        body,
        grid=(nb,),
        in_specs=[
            pl.BlockSpec((RB, dim), lambda i: (i, 0)),
            pl.BlockSpec((8, dim), lambda i: (0, 0)),
            pl.BlockSpec((8, dim), lambda i: (0, 0)),
            pl.BlockSpec((1, dim), lambda i: (0, 0)),
            pl.BlockSpec((1, dim), lambda i: (0, 0)),
            pl.BlockSpec((dim, dim), lambda i: (0, 0)),
            pl.BlockSpec((1, dim), lambda i: (0, 0)),
        ],
        out_specs=[
            pl.BlockSpec((RB, dim), lambda i: (i, 0)),
            pl.BlockSpec((8, dim), lambda i: (0, 0)),
            pl.BlockSpec((8, dim), lambda i: (0, 0)),
        ],
        out_shape=[
            jax.ShapeDtypeStruct((RN, dim), _f32),
            jax.ShapeDtypeStruct((8, dim), _f32),
            jax.ShapeDtypeStruct((8, dim), _f32),
        ],
        name="gin_mm2",
    )(y1, s1, q1, g1, bb1, w2, b2)


def _bn_relu_mean(y2, s2, q2, g2, bb2, n):
    RN, dim = y2.shape
    W = dim // 4
    nb = n // RB
    inv_n = 1.0 / RN
    inv_r = 1.0 / NUM_RUNS

    def body(y_ref, s_ref, q_ref, g_ref, bb_ref, h0, h1, h2, h3, m_ref):
        r = pl.program_id(1)
        mu = s_ref[...].sum(axis=0, keepdims=True) * inv_n
        var = q_ref[...].sum(axis=0, keepdims=True) * inv_n - mu * mu
        scale = g_ref[...] * lax.rsqrt(var + 1e-5)
        h = jnp.maximum((y_ref[...] - mu) * scale + bb_ref[...], 0.0)
        outs = [h0, h1, h2, h3]
        for j in range(4):
            outs[j][...] = h[:, j * W:(j + 1) * W]

        @pl.when(r == 0)
        def _():
            m_ref[...] = h * inv_r

        @pl.when(r > 0)
        def _():
            m_ref[...] += h * inv_r

    return pl.pallas_call(
        body,
        grid=(nb, NUM_RUNS),
        in_specs=[
            pl.BlockSpec((RB, dim), lambda i, r: (r * (RN // NUM_RUNS // RB) + i, 0)),
            pl.BlockSpec((8, dim), lambda i, r: (0, 0)),
            pl.BlockSpec((8, dim), lambda i, r: (0, 0)),
            pl.BlockSpec((1, dim), lambda i, r: (0, 0)),
            pl.BlockSpec((1, dim), lambda i, r: (0, 0)),
        ],
        out_specs=[pl.BlockSpec((RB, W),
                                lambda i, r: (r * (RN // NUM_RUNS // RB) + i, 0))
                   for _ in range(4)] +
                  [pl.BlockSpec((RB, dim), lambda i, r: (i, 0))],
        out_shape=[jax.ShapeDtypeStruct((RN, W), _f32)] * 4 +
                  [jax.ShapeDtypeStruct((n, dim), _f32)],
        name="gin_bn_mean",
    )(y2, s2, q2, g2, bb2)


def _readout(x, kf, ms, wcat, bsum):
    n, fin = x.shape
    nb = n // RB
    c = wcat.shape[1]
    kdim = wcat.shape[0]

    def body(x_ref, kf_ref, m1, m2, m3, m4, w_ref, b_ref, o_ref):
        m0 = x_ref[...] * kf_ref[...]
        mcat = jnp.concatenate(
            [m0, m1[...], m2[...], m3[...], m4[...]], axis=1)
        logits = lax.dot_general(mcat, w_ref[...], (((1,), (0,)), ((), ())),
                                 preferred_element_type=_f32) + b_ref[...]
        mx = jnp.max(logits, axis=1, keepdims=True)
        sh = logits - mx
        lse = jnp.log(jnp.sum(jnp.exp(sh), axis=1, keepdims=True))
        o_ref[...] = sh - lse

    dim = ms[0].shape[1]
    return pl.pallas_call(
        body,
        grid=(nb,),
        in_specs=[
            pl.BlockSpec((RB, fin), lambda i: (i, 0)),
            pl.BlockSpec((RB, 1), lambda i: (i, 0)),
        ] + [pl.BlockSpec((RB, dim), lambda i: (i, 0)) for _ in range(4)] + [
            pl.BlockSpec((kdim, c), lambda i: (0, 0)),
            pl.BlockSpec((1, c), lambda i: (0, 0)),
        ],
        out_specs=pl.BlockSpec((RB, c), lambda i: (i, 0)),
        out_shape=jax.ShapeDtypeStruct((n, c), _f32),
        name="gin_readout",
    )(x, kf, *ms, wcat, bsum)


def kernel(x, edge_index, batch, params):
    convs, bns, fcs = params
    n, fin = x.shape
    R = NUM_RUNS
    RN = R * n
    num_layers = len(convs)

    drop = jax.random.bernoulli(jax.random.key(42), P_DROP, (R, n))
    keep = (1.0 - drop.astype(_f32)).reshape(R, n, 1)
    kf = keep.mean(axis=0)

    src = edge_index[0]
    dst = edge_index[1]
    offset = jnp.max(edge_index) + 1
    run_off = (jnp.arange(R, dtype=edge_index.dtype)[:, None] * offset)
    rsrc = (src[None, :] + run_off).reshape(-1)
    rdst = (dst[None, :] + run_off).reshape(-1)
    RE = rsrc.shape[0]
    RE_pad = ((RE + NTEC * EB - 1) // (NTEC * EB)) * (NTEC * EB)
    if RE_pad != RE:
        pad = RE_pad - RE
        rsrc = jnp.concatenate([rsrc, jnp.zeros((pad,), rsrc.dtype)])
        rdst = jnp.concatenate([rdst, jnp.full((pad,), RN, rdst.dtype)])

    dump_rows = NTEC * EB
    nz = (RN // NSC + dump_rows + NTEC * 8 - 1) // (NTEC * 8) * 8
    zrows32 = jnp.zeros((nz, fin // 4), _f32)
    zrows64 = None

    hs = _drop_expand(x, keep, fin // 4)

    ms = []
    for i in range(num_layers):
        w1, b1, g1, bb1, w2, b2 = convs[i]
        g, b = bns[i]
        W = hs[0].shape[1]
        if W == fin // 4:
            zr = zrows32
        else:
            if zrows64 is None:
                zrows64 = jnp.zeros((nz, W), _f32)
            zr = zrows64
        seg = _make_segment_sum(RN, W, RE_pad, dump_rows)
        aggs = seg(rsrc, rdst, zr, *hs)
        y1, s1, q1 = _mm1_stats(hs, aggs, w1, b1.reshape(1, -1))
        y2, s2, q2 = _bn_relu_mm2(y1, s1, q1, g1.reshape(1, -1),
                                  bb1.reshape(1, -1), w2, b2.reshape(1, -1))
        *hs, m = _bn_relu_mean(y2, s2, q2, g.reshape(1, -1),
                               b.reshape(1, -1), n)
        ms.append(m)

    wcat = jnp.concatenate([w for (w, _) in fcs], axis=0)
    bsum = sum(bb for (_, bb) in fcs).reshape(1, -1)
    return _readout(x, kf, ms, wcat, bsum)
